# Initial kernel scaffold; baseline (speedup 1.0000x reference)
import jax
import jax.numpy as jnp
from jax.experimental import pallas as pl


def kernel(seed, W0, W1, W2, W3):
    raise NotImplementedError("write your pallas kernel here")


# speedup vs baseline: 7.0312x; 7.0312x over previous
import functools

import jax
import jax.numpy as jnp
import numpy as np
from jax import lax
from jax.experimental import pallas as pl
from jax.experimental.pallas import tpu as pltpu
from jax.experimental.pallas import tpu_sc as plsc

_N = 8192
_D = 32
_K = 16
_RATE = np.float32(0.0001)

_BR = 256


def _knn_body(pxc_ref, pyc_ref, pxr_ref, pyr_ref, out_ref):
    pxc = pxc_ref[...]
    pyc = pyc_ref[...]
    pxr = pxr_ref[...]
    pyr = pyr_ref[...]
    dx = pxc - pxr
    dy = pyc - pyr
    acc = dx * dx + dy * dy
    iota = lax.broadcasted_iota(jnp.int32, (_BR, _N), 1)
    cols = []
    for _ in range(_K):
        m = jnp.min(acc, axis=1, keepdims=True)
        sel = jnp.min(jnp.where(acc <= m, iota, _N), axis=1, keepdims=True)
        cols.append(sel)
        acc = jnp.where(iota == sel, jnp.float32(np.inf), acc)
    out_ref[...] = jnp.concatenate(cols, axis=1)


def _knn(pos):
    px = pos[:, 0]
    py = pos[:, 1]
    pxc = px.reshape(_N, 1)
    pyc = py.reshape(_N, 1)
    pxr = px.reshape(1, _N)
    pyr = py.reshape(1, _N)
    return pl.pallas_call(
        _knn_body,
        grid=(_N // _BR,),
        in_specs=[
            pl.BlockSpec((_BR, 1), lambda i: (i, 0)),
            pl.BlockSpec((_BR, 1), lambda i: (i, 0)),
            pl.BlockSpec((1, _N), lambda i: (0, 0)),
            pl.BlockSpec((1, _N), lambda i: (0, 0)),
        ],
        out_specs=pl.BlockSpec((_BR, _K), lambda i: (i, 0)),
        out_shape=jax.ShapeDtypeStruct((_N, _K), jnp.int32),
    )(pxc, pyc, pxr, pyr)


_BM = 1024


def _mm_body(h_ref, w_ref, o_ref, *, relu, scale):
    h = h_ref[...]
    if relu:
        h = jnp.maximum(h, jnp.float32(0.0))
    o = jnp.dot(h, w_ref[...], preferred_element_type=jnp.float32)
    if scale is not None:
        o = o * scale
    o_ref[...] = o


def _mm(h, w, relu, scale=None):
    din = h.shape[1]
    dout = w.shape[1]
    return pl.pallas_call(
        functools.partial(_mm_body, relu=relu, scale=scale),
        grid=(_N // _BM,),
        in_specs=[
            pl.BlockSpec((_BM, din), lambda i: (i, 0)),
            pl.BlockSpec((din, dout), lambda i: (0, 0)),
        ],
        out_specs=pl.BlockSpec((_BM, dout), lambda i: (i, 0)),
        out_shape=jax.ShapeDtypeStruct((_N, dout), jnp.float32),
    )(h, w)


_NC = 2
_NS = 16
_NW = _NC * _NS
_RPW = _N // _NW
_RC = 64
_CNT = _RC * _K


def _gather_mean(y, idx_flat, xres):
    dim = y.shape[1]
    nlanes = dim // 16
    with_res = xres is not None
    mesh = plsc.VectorSubcoreMesh(core_axis_name="c", subcore_axis_name="s")

    scratch = [
        pltpu.VMEM((_CNT,), jnp.int32),
        pltpu.VMEM((_CNT, dim), jnp.float32),
        pltpu.VMEM((_RC, dim), jnp.float32),
        pltpu.SemaphoreType.DMA,
    ]
    if with_res:
        scratch.append(pltpu.VMEM((_RC, dim), jnp.float32))

    def body(y_hbm, idx_hbm, *rest):
        if with_res:
            x_hbm, out_hbm, idx_v, rows_v, acc_v, sem, xr_v = rest
        else:
            out_hbm, idx_v, rows_v, acc_v, sem = rest
        wid = lax.axis_index("s") * _NC + lax.axis_index("c")
        inv_k = jnp.float32(1.0 / _K)

        for g in range(_RPW // _RC):
            row_base = wid * _RPW + g * _RC
            pltpu.sync_copy(idx_hbm.at[pl.ds(row_base * _K, _CNT)], idx_v)
            pltpu.async_copy(y_hbm.at[idx_v], rows_v, sem).wait()
            if with_res:
                pltpu.sync_copy(x_hbm.at[pl.ds(row_base, _RC)], xr_v)

            def acc_row(r, carry):
                for c in range(nlanes):
                    tot = rows_v[r * _K, pl.ds(c * 16, 16)]
                    for j in range(1, _K):
                        tot = tot + rows_v[r * _K + j, pl.ds(c * 16, 16)]
                    tot = tot * inv_k
                    if with_res:
                        tot = tot + xr_v[r, pl.ds(c * 16, 16)]
                    acc_v[r, pl.ds(c * 16, 16)] = tot
                return carry

            lax.fori_loop(0, _RC, acc_row, 0)
            pltpu.sync_copy(acc_v, out_hbm.at[pl.ds(row_base, _RC)])

    args = (y, idx_flat) + ((xres,) if with_res else ())
    return pl.kernel(
        body,
        mesh=mesh,
        out_type=jax.ShapeDtypeStruct((_N, dim), jnp.float32),
        scratch_types=scratch,
        compiler_params=pltpu.CompilerParams(use_tc_tiling_on_sc=False),
    )(*args)


def kernel(seed, W0, W1, W2, W3):
    x = seed
    idx = _knn(x[:, :2])
    idx_flat = idx.reshape(_N * _K)

    y0 = _mm(x, W0, relu=False)
    h1 = _gather_mean(y0, idx_flat, None)
    y1 = _mm(h1, W1, relu=True)
    h2 = _gather_mean(y1, idx_flat, None)
    y2 = _mm(h2, W2, relu=True)
    h3 = _gather_mean(y2, idx_flat, None)
    y3 = _mm(h3, W3, relu=True, scale=_RATE)
    out = _gather_mean(y3, idx_flat, x)
    return out
